# baseline (speedup 1.0000x reference)
import jax
import jax.numpy as jnp
from jax import lax
from jax.experimental import pallas as pl
from jax.experimental.pallas import tpu as pltpu
from jax.experimental.pallas import tpu_sc as plsc

NC = 2
NS = 16
NW = NC * NS
CH = 128
NBUF = 2
L = 16
DUNROLL = 16


def _transpose_chunk(buf, tbuf, D):
  iota = lax.iota(jnp.int32, L)
  lidx = [iota + L * m for m in range(CH // L)]
  zero = iota * 0

  @pl.loop(0, D, step=DUNROLL)
  def _td(d0):
    base0 = d0 * CH
    d0v = zero + d0
    for dd in range(DUNROLL):
      didx = d0v + dd
      for m in range(CH // L):
        v = plsc.load_gather(buf, [lidx[m], didx])
        tbuf[pl.ds(base0 + (dd * CH + L * m), L)] = v


def _make_body(B, H, D):
  ncn = B // (NW * CH)
  nce = (B * H) // (NW * CH)
  assert nce % NBUF == 0
  tpd = D // 8
  cph = B // CH

  def body(nodes_hbm, neigh_hbm, user_hbm, item_hbm,
           node_out, neigh_out, idx_n_v, idx_e_v, *scratch):
    bufs = scratch[:NBUF]
    tbufs = scratch[NBUF:2 * NBUF]
    gsem = scratch[2 * NBUF:3 * NBUF]
    ssem = scratch[3 * NBUF:4 * NBUF]

    w = lax.axis_index("s") * NC + lax.axis_index("c")
    pltpu.sync_copy(nodes_hbm.at[w], idx_n_v)
    pltpu.sync_copy(neigh_hbm.at[w], idx_e_v)

    def store_chunk(tbuf, out, g, sem):
      mbase = ((g // cph) * tpd) * cph + (g % cph)
      for i in range(tpd):
        pltpu.async_copy(tbuf.at[pl.ds(i * 1024, 1024)],
                         out.at[pl.ds((mbase + i * cph) * 1024, 1024)], sem)

    def wait_store(tbuf, out, g, sem):
      mbase = ((g // cph) * tpd) * cph + (g % cph)
      for i in range(tpd):
        pltpu.make_async_copy(
            tbuf.at[pl.ds(i * 1024, 1024)],
            out.at[pl.ds((mbase + i * cph) * 1024, 1024)], sem).wait()

    def ring(table_hbm, idx_v, out, nchunk, gbase):
      for b in range(NBUF):
        pltpu.async_copy(table_hbm.at[idx_v.at[b]], bufs[b], gsem[b])

      @pl.loop(0, nchunk, step=NBUF)
      def _round(c0):
        for b in range(NBUF):
          c = c0 + b
          pltpu.make_async_copy(table_hbm.at[idx_v.at[c]], bufs[b],
                                gsem[b]).wait()
          _transpose_chunk(bufs[b], tbufs[b], D)
          store_chunk(tbufs[b], out, gbase + c, ssem[b])
        for b in range(NBUF):
          c = c0 + b
          wait_store(tbufs[b], out, gbase + c, ssem[b])
          cn = lax.rem(c + NBUF, nchunk)
          pltpu.async_copy(table_hbm.at[idx_v.at[cn]], bufs[b], gsem[b])

      for b in range(NBUF):
        pltpu.make_async_copy(table_hbm.at[idx_v.at[b]], bufs[b],
                              gsem[b]).wait()

    ring(user_hbm, idx_n_v, node_out, ncn, w * ncn)
    ring(item_hbm, idx_e_v, neigh_out, nce, w * nce)

  return body, ncn, nce


def kernel(nodes, neighbors, degrees, user_table, item_table):
  B, H = neighbors.shape
  D = user_table.shape[1]
  assert B % (NW * CH) == 0 and (B * H) % (NW * CH) == 0 and D % 8 == 0

  body, ncn, nce = _make_body(B, H, D)
  tpd = D // 8

  mesh = plsc.VectorSubcoreMesh(
      core_axis_name="c", subcore_axis_name="s",
      num_cores=NC, num_subcores=NS)

  scratch = ([pltpu.VMEM((ncn, CH), jnp.int32),
              pltpu.VMEM((nce, CH), jnp.int32)]
             + [pltpu.VMEM((CH, D), jnp.float32) for _ in range(NBUF)]
             + [pltpu.VMEM((CH * D,), jnp.float32) for _ in range(NBUF)]
             + [pltpu.SemaphoreType.DMA for _ in range(2 * NBUF)])

  run = pl.kernel(
      body,
      out_type=(
          jax.ShapeDtypeStruct((B * D,), user_table.dtype),
          jax.ShapeDtypeStruct((B * H * D,), item_table.dtype),
      ),
      mesh=mesh,
      compiler_params=pltpu.CompilerParams(
          use_tc_tiling_on_sc=False, needs_layout_passes=False),
      scratch_types=scratch,
  )

  nodes_r = nodes.astype(jnp.int32).reshape(NW, ncn, CH)
  neigh_r = neighbors.astype(jnp.int32).T.reshape(NW, nce, CH)
  node_flat, neigh_flat = run(nodes_r, neigh_r, user_table, item_table)

  node_emb = (node_flat.reshape(tpd, B // CH, 8, CH)
              .transpose(1, 3, 0, 2).reshape(B, D))
  neigh_emb = (neigh_flat.reshape(H, tpd, B // CH, 8, CH)
               .transpose(2, 4, 0, 1, 3).reshape(B, H, D))
  return (node_emb, neigh_emb, degrees)


# speedup vs baseline: 1.4660x; 1.4660x over previous
import jax
import jax.numpy as jnp
from jax import lax
from jax.experimental import pallas as pl
from jax.experimental.pallas import tpu as pltpu
from jax.experimental.pallas import tpu_sc as plsc

NC = 2
NS = 16
NW = NC * NS
CH = 128
NBUF = 2
L = 16
def _transpose_chunk(buf, tbuf, D):
  iota = lax.iota(jnp.int32, L)
  rot = [lax.rem(iota + k, L) for k in range(L)]
  rotw = [rot[k] * CH + iota for k in range(L)]

  @pl.loop(0, D, step=L)
  def _td(d0):
    didx = [rot[k] + d0 for k in range(L)]
    for l0 in range(0, CH, L):
      lidx = iota + l0
      wbase = d0 * CH + l0
      for k in range(L):
        v = plsc.load_gather(buf, [lidx, didx[k]])
        plsc.store_scatter(tbuf, [rotw[k] + wbase], v)


def _make_body(B, H, D):
  ncn = B // (NW * CH)
  nce = (B * H) // (NW * CH)
  assert nce % NBUF == 0
  tpd = D // 8
  cph = B // CH

  def body(nodes_hbm, neigh_hbm, user_hbm, item_hbm,
           node_out, neigh_out, idx_n_v, idx_e_v, *scratch):
    bufs = scratch[:NBUF]
    tbufs = scratch[NBUF:2 * NBUF]
    gsem = scratch[2 * NBUF:3 * NBUF]
    ssem = scratch[3 * NBUF:4 * NBUF]

    w = lax.axis_index("s") * NC + lax.axis_index("c")
    pltpu.sync_copy(nodes_hbm.at[w], idx_n_v)
    pltpu.sync_copy(neigh_hbm.at[w], idx_e_v)

    def store_chunk(tbuf, out, g, sem):
      mbase = ((g // cph) * tpd) * cph + (g % cph)
      for i in range(tpd):
        pltpu.async_copy(tbuf.at[pl.ds(i * 1024, 1024)],
                         out.at[pl.ds((mbase + i * cph) * 1024, 1024)], sem)

    def wait_store(tbuf, out, g, sem):
      mbase = ((g // cph) * tpd) * cph + (g % cph)
      for i in range(tpd):
        pltpu.make_async_copy(
            tbuf.at[pl.ds(i * 1024, 1024)],
            out.at[pl.ds((mbase + i * cph) * 1024, 1024)], sem).wait()

    def ring(table_hbm, idx_v, out, nchunk, gbase):
      for b in range(NBUF):
        pltpu.async_copy(table_hbm.at[idx_v.at[b]], bufs[b], gsem[b])

      @pl.loop(0, nchunk, step=NBUF)
      def _round(c0):
        for b in range(NBUF):
          c = c0 + b
          pltpu.make_async_copy(table_hbm.at[idx_v.at[c]], bufs[b],
                                gsem[b]).wait()
          _transpose_chunk(bufs[b], tbufs[b], D)
          store_chunk(tbufs[b], out, gbase + c, ssem[b])
        for b in range(NBUF):
          c = c0 + b
          wait_store(tbufs[b], out, gbase + c, ssem[b])
          cn = lax.rem(c + NBUF, nchunk)
          pltpu.async_copy(table_hbm.at[idx_v.at[cn]], bufs[b], gsem[b])

      for b in range(NBUF):
        pltpu.make_async_copy(table_hbm.at[idx_v.at[b]], bufs[b],
                              gsem[b]).wait()

    ring(user_hbm, idx_n_v, node_out, ncn, w * ncn)
    ring(item_hbm, idx_e_v, neigh_out, nce, w * nce)

  return body, ncn, nce


def kernel(nodes, neighbors, degrees, user_table, item_table):
  B, H = neighbors.shape
  D = user_table.shape[1]
  assert B % (NW * CH) == 0 and (B * H) % (NW * CH) == 0 and D % 8 == 0

  body, ncn, nce = _make_body(B, H, D)
  tpd = D // 8

  mesh = plsc.VectorSubcoreMesh(
      core_axis_name="c", subcore_axis_name="s",
      num_cores=NC, num_subcores=NS)

  scratch = ([pltpu.VMEM((ncn, CH), jnp.int32),
              pltpu.VMEM((nce, CH), jnp.int32)]
             + [pltpu.VMEM((CH, D), jnp.float32) for _ in range(NBUF)]
             + [pltpu.VMEM((CH * D,), jnp.float32) for _ in range(NBUF)]
             + [pltpu.SemaphoreType.DMA for _ in range(2 * NBUF)])

  run = pl.kernel(
      body,
      out_type=(
          jax.ShapeDtypeStruct((B * D,), user_table.dtype),
          jax.ShapeDtypeStruct((B * H * D,), item_table.dtype),
      ),
      mesh=mesh,
      compiler_params=pltpu.CompilerParams(
          use_tc_tiling_on_sc=False, needs_layout_passes=False),
      scratch_types=scratch,
  )

  nodes_r = nodes.astype(jnp.int32).reshape(NW, ncn, CH)
  neigh_r = neighbors.astype(jnp.int32).T.reshape(NW, nce, CH)
  node_flat, neigh_flat = run(nodes_r, neigh_r, user_table, item_table)

  node_emb = (node_flat.reshape(tpd, B // CH, 8, CH)
              .transpose(1, 3, 0, 2).reshape(B, D))
  neigh_emb = (neigh_flat.reshape(H, tpd, B // CH, 8, CH)
               .transpose(2, 4, 0, 1, 3).reshape(B, H, D))
  return (node_emb, neigh_emb, degrees)


# speedup vs baseline: 1.4716x; 1.0038x over previous
import jax
import jax.numpy as jnp
from jax import lax
from jax.experimental import pallas as pl
from jax.experimental.pallas import tpu as pltpu
from jax.experimental.pallas import tpu_sc as plsc

NC = 2
NS = 16
NW = NC * NS
CH = 128
NBUF = 4
L = 16
def _transpose_chunk(buf, tbuf, D):
  iota = lax.iota(jnp.int32, L)
  rot = [lax.rem(iota + k, L) for k in range(L)]
  rotw = [rot[k] * CH + iota for k in range(L)]

  @pl.loop(0, D, step=L)
  def _td(d0):
    didx = [rot[k] + d0 for k in range(L)]
    for l0 in range(0, CH, L):
      lidx = iota + l0
      wbase = d0 * CH + l0
      for k in range(L):
        v = plsc.load_gather(buf, [lidx, didx[k]])
        plsc.store_scatter(tbuf, [rotw[k] + wbase], v)


def _make_body(B, H, D):
  ncn = B // (NW * CH)
  nce = (B * H) // (NW * CH)
  assert nce % NBUF == 0
  tpd = D // 8
  cph = B // CH

  def body(nodes_hbm, neigh_hbm, user_hbm, item_hbm,
           node_out, neigh_out, idx_n_v, idx_e_v, *scratch):
    bufs = scratch[:NBUF]
    tbufs = scratch[NBUF:2 * NBUF]
    gsem = scratch[2 * NBUF:3 * NBUF]
    ssem = scratch[3 * NBUF:4 * NBUF]

    w = lax.axis_index("s") * NC + lax.axis_index("c")
    pltpu.sync_copy(nodes_hbm.at[w], idx_n_v)
    pltpu.sync_copy(neigh_hbm.at[w], idx_e_v)

    def store_chunk(tbuf, out, g, sem):
      mbase = ((g // cph) * tpd) * cph + (g % cph)
      for i in range(tpd):
        pltpu.async_copy(tbuf.at[pl.ds(i * 1024, 1024)],
                         out.at[pl.ds((mbase + i * cph) * 1024, 1024)], sem)

    def wait_store(tbuf, out, g, sem):
      mbase = ((g // cph) * tpd) * cph + (g % cph)
      for i in range(tpd):
        pltpu.make_async_copy(
            tbuf.at[pl.ds(i * 1024, 1024)],
            out.at[pl.ds((mbase + i * cph) * 1024, 1024)], sem).wait()

    def ring(table_hbm, idx_v, out, nchunk, gbase):
      for b in range(NBUF):
        pltpu.async_copy(table_hbm.at[idx_v.at[b]], bufs[b], gsem[b])

      @pl.loop(0, nchunk, step=NBUF)
      def _round(c0):
        for b in range(NBUF):
          c = c0 + b
          pltpu.make_async_copy(table_hbm.at[idx_v.at[c]], bufs[b],
                                gsem[b]).wait()
          _transpose_chunk(bufs[b], tbufs[b], D)
          store_chunk(tbufs[b], out, gbase + c, ssem[b])
        for b in range(NBUF):
          c = c0 + b
          wait_store(tbufs[b], out, gbase + c, ssem[b])
          cn = lax.rem(c + NBUF, nchunk)
          pltpu.async_copy(table_hbm.at[idx_v.at[cn]], bufs[b], gsem[b])

      for b in range(NBUF):
        pltpu.make_async_copy(table_hbm.at[idx_v.at[b]], bufs[b],
                              gsem[b]).wait()

    ring(user_hbm, idx_n_v, node_out, ncn, w * ncn)
    ring(item_hbm, idx_e_v, neigh_out, nce, w * nce)

  return body, ncn, nce


def kernel(nodes, neighbors, degrees, user_table, item_table):
  B, H = neighbors.shape
  D = user_table.shape[1]
  assert B % (NW * CH) == 0 and (B * H) % (NW * CH) == 0 and D % 8 == 0

  body, ncn, nce = _make_body(B, H, D)
  tpd = D // 8

  mesh = plsc.VectorSubcoreMesh(
      core_axis_name="c", subcore_axis_name="s",
      num_cores=NC, num_subcores=NS)

  scratch = ([pltpu.VMEM((ncn, CH), jnp.int32),
              pltpu.VMEM((nce, CH), jnp.int32)]
             + [pltpu.VMEM((CH, D), jnp.float32) for _ in range(NBUF)]
             + [pltpu.VMEM((CH * D,), jnp.float32) for _ in range(NBUF)]
             + [pltpu.SemaphoreType.DMA for _ in range(2 * NBUF)])

  run = pl.kernel(
      body,
      out_type=(
          jax.ShapeDtypeStruct((B * D,), user_table.dtype),
          jax.ShapeDtypeStruct((B * H * D,), item_table.dtype),
      ),
      mesh=mesh,
      compiler_params=pltpu.CompilerParams(
          use_tc_tiling_on_sc=False, needs_layout_passes=False),
      scratch_types=scratch,
  )

  nodes_r = nodes.astype(jnp.int32).reshape(NW, ncn, CH)
  neigh_r = neighbors.astype(jnp.int32).T.reshape(NW, nce, CH)
  node_flat, neigh_flat = run(nodes_r, neigh_r, user_table, item_table)

  node_emb = (node_flat.reshape(tpd, B // CH, 8, CH)
              .transpose(1, 3, 0, 2).reshape(B, D))
  neigh_emb = (neigh_flat.reshape(H, tpd, B // CH, 8, CH)
               .transpose(2, 4, 0, 1, 3).reshape(B, H, D))
  return (node_emb, neigh_emb, degrees)


# speedup vs baseline: 1.4775x; 1.0040x over previous
import jax
import jax.numpy as jnp
from jax import lax
from jax.experimental import pallas as pl
from jax.experimental.pallas import tpu as pltpu
from jax.experimental.pallas import tpu_sc as plsc

NC = 2
NS = 16
NW = NC * NS
CH = 128
NBUF = 8


def _make_body(B, H, D):
  ncn = B // (NW * CH)
  nce = (B * H) // (NW * CH)
  assert nce % NBUF == 0
  rows_n = ncn * CH
  rows_e = nce * CH

  def body(nodes_hbm, neigh_hbm, user_hbm, item_hbm,
           node_out, neigh_out, idx_n_v, idx_e_v, *scratch):
    bufs = scratch[:NBUF]
    gsem = scratch[NBUF:2 * NBUF]
    ssem = scratch[2 * NBUF:3 * NBUF]

    w = lax.axis_index("s") * NC + lax.axis_index("c")
    pltpu.sync_copy(nodes_hbm.at[w], idx_n_v)
    pltpu.sync_copy(neigh_hbm.at[w], idx_e_v)
    nbase = w * rows_n
    ebase = w * rows_e

    for j in range(ncn):
      pltpu.async_copy(user_hbm.at[idx_n_v.at[j]], bufs[j % NBUF],
                       gsem[j % NBUF])
    for j in range(ncn):
      b = j % NBUF
      pltpu.make_async_copy(user_hbm.at[idx_n_v.at[j]], bufs[b],
                            gsem[b]).wait()
      pltpu.async_copy(bufs[b], node_out.at[pl.ds(nbase + j * CH, CH)],
                       ssem[b])
    for j in range(ncn):
      b = j % NBUF
      pltpu.make_async_copy(bufs[b], node_out.at[pl.ds(nbase + j * CH, CH)],
                            ssem[b]).wait()

    for b in range(NBUF):
      pltpu.async_copy(item_hbm.at[idx_e_v.at[b]], bufs[b], gsem[b])

    def _do_round(c0, refill):
      for b in range(NBUF):
        c = c0 + b
        pltpu.make_async_copy(item_hbm.at[idx_e_v.at[c]], bufs[b],
                              gsem[b]).wait()
        pltpu.async_copy(bufs[b], neigh_out.at[pl.ds(ebase + c * CH, CH)],
                         ssem[b])
      for b in range(NBUF):
        c = c0 + b
        pltpu.make_async_copy(bufs[b],
                              neigh_out.at[pl.ds(ebase + c * CH, CH)],
                              ssem[b]).wait()
        if refill:
          pltpu.async_copy(item_hbm.at[idx_e_v.at[c + NBUF]], bufs[b],
                           gsem[b])

    @pl.loop(0, nce - NBUF, step=NBUF)
    def _round(c0):
      _do_round(c0, True)

    _do_round(nce - NBUF, False)

  return body, ncn, nce


def kernel(nodes, neighbors, degrees, user_table, item_table):
  B, H = neighbors.shape
  D = user_table.shape[1]
  assert B % (NW * CH) == 0 and (B * H) % (NW * CH) == 0

  body, ncn, nce = _make_body(B, H, D)

  mesh = plsc.VectorSubcoreMesh(
      core_axis_name="c", subcore_axis_name="s",
      num_cores=NC, num_subcores=NS)

  scratch = ([pltpu.VMEM((ncn, CH), jnp.int32),
              pltpu.VMEM((nce, CH), jnp.int32)]
             + [pltpu.VMEM((CH, D), jnp.float32) for _ in range(NBUF)]
             + [pltpu.SemaphoreType.DMA for _ in range(2 * NBUF)])

  run = pl.kernel(
      body,
      out_type=(
          jax.ShapeDtypeStruct((B, D), user_table.dtype),
          jax.ShapeDtypeStruct((B * H, D), item_table.dtype),
      ),
      mesh=mesh,
      compiler_params=pltpu.CompilerParams(use_tc_tiling_on_sc=False),
      scratch_types=scratch,
  )

  nodes_r = nodes.astype(jnp.int32).reshape(NW, ncn, CH)
  neigh_r = neighbors.astype(jnp.int32).reshape(NW, nce, CH)
  node_emb, neigh_flat = run(nodes_r, neigh_r, user_table, item_table)
  return (node_emb, neigh_flat.reshape(B, H, D), degrees)
